# Initial kernel scaffold; baseline (speedup 1.0000x reference)
import jax
import jax.numpy as jnp
from jax.experimental import pallas as pl


def kernel(data, edge_index, W1, b1, g1, be1, W2, b2, g2, be2, W3, b3):
    raise NotImplementedError("write your pallas kernel here")


# speedup vs baseline: 14.7895x; 14.7895x over previous
import functools

import jax
import jax.numpy as jnp
from jax import lax
from jax.experimental import pallas as pl
from jax.experimental.pallas import tpu as pltpu
from jax.experimental.pallas import tpu_sc as plsc

N = 50000
NP = 50048
E = 800000
ERP = 6400
EP = ERP * 128
RT2 = ERP // 32
RT1 = ERP // 16
EPS = 1e-5
BN = 2944
GRID = NP // BN
TS = NP // 16
F32 = jnp.float32

_MESH = plsc.VectorSubcoreMesh(
    core_axis_name="c", subcore_axis_name="s", num_cores=2, num_subcores=16)


@functools.partial(
    pl.kernel,
    out_type=jax.ShapeDtypeStruct((32 * NP,), F32),
    mesh=_MESH,
    scratch_types=[
        pltpu.VMEM((RT2, 128), jnp.int32),
        pltpu.VMEM((NP,), F32),
    ],
    compiler_params=pltpu.CompilerParams(needs_layout_passes=False),
)
def _sc_deg(dst_hbm, out_hbm, dstv, acc):
    c = lax.axis_index("c")
    s = lax.axis_index("s")
    w = c * 16 + s

    @pl.loop(0, NP // 16)
    def _zero(i):
        acc[pl.ds(i * 16, 16)] = jnp.zeros((16,), F32)

    pltpu.sync_copy(dst_hbm.at[pl.ds(w * RT2, RT2)], dstv)
    ones = jnp.ones((16,), F32)

    @pl.loop(0, RT2)
    def _edges(j):
        for b in range(8):
            plsc.addupdate_scatter(acc, [dstv[j, pl.ds(b * 16, 16)]], ones)

    pltpu.sync_copy(acc, out_hbm.at[pl.ds(w * NP, NP)])


@functools.partial(
    pl.kernel,
    out_type=jax.ShapeDtypeStruct((32 * NP,), F32),
    mesh=_MESH,
    scratch_types=[
        pltpu.VMEM((8, 128), jnp.int32),
        pltpu.VMEM((8, 128), jnp.int32),
        pltpu.VMEM((NP,), F32),
        pltpu.VMEM((NP,), F32),
    ],
    compiler_params=pltpu.CompilerParams(needs_layout_passes=False),
)
def _sc_agg_scalar(src_hbm, dst_hbm, y_hbm, out_hbm, srcv, dstv, yv, acc):
    c = lax.axis_index("c")
    s = lax.axis_index("s")
    w = c * 16 + s

    @pl.loop(0, NP // 16)
    def _zero(i):
        acc[pl.ds(i * 16, 16)] = jnp.zeros((16,), F32)

    pltpu.sync_copy(y_hbm, yv)

    @pl.loop(0, RT2 // 8)
    def _chunk(k):
        rsl = pl.ds(w * RT2 + k * 8, 8)
        pltpu.sync_copy(src_hbm.at[rsl], srcv)
        pltpu.sync_copy(dst_hbm.at[rsl], dstv)

        @pl.loop(0, 8)
        def _row(j):
            for b in range(8):
                lane = pl.ds(b * 16, 16)
                vals = plsc.load_gather(yv, [srcv[j, lane]])
                plsc.addupdate_scatter(acc, [dstv[j, lane]], vals)

    pltpu.sync_copy(acc, out_hbm.at[pl.ds(w * NP, NP)])


_QSHAPE = jax.ShapeDtypeStruct((NP, 16), F32)


@functools.partial(
    pl.kernel,
    out_type=(_QSHAPE, _QSHAPE, _QSHAPE, _QSHAPE),
    mesh=_MESH,
    scratch_types=[
        pltpu.VMEM((40, 128), jnp.int32),
        pltpu.VMEM((40, 128), jnp.int32),
        pltpu.VMEM((128, 16), F32),
        pltpu.VMEM((136, 16), F32),
        pltpu.VMEM_SHARED((NP, 16), F32),
    ],
    compiler_params=pltpu.CompilerParams(use_tc_tiling_on_sc=False),
)
def _sc_agg_wide(z0_hbm, z1_hbm, z2_hbm, z3_hbm, src_hbm, dst_hbm,
                 o0_hbm, o1_hbm, o2_hbm, o3_hbm,
                 srcv, dstv, rowv, bounce, acc):
    c = lax.axis_index("c")
    s = lax.axis_index("s")

    def quarter(z_ref, out_ref):
        @pl.loop(0, TS // 136)
        def _init(k):
            csl = pl.ds(s * TS + k * 136, 136)
            pltpu.sync_copy(z_ref.at[csl], bounce)
            pltpu.sync_copy(bounce, acc.at[csl])

        plsc.subcore_barrier()

        @pl.loop(0, RT1 // 40)
        def _chunk(k):
            rsl = pl.ds(s * RT1 + k * 40, 40)
            pltpu.sync_copy(src_hbm.at[rsl], srcv)
            pltpu.sync_copy(dst_hbm.at[rsl], dstv)

            @pl.loop(0, 40)
            def _edges(j):
                pltpu.sync_copy(z_ref.at[srcv.at[j]], rowv)
                pltpu.sync_copy(rowv, acc.at[dstv.at[j]], add=True)

        plsc.subcore_barrier()

        @pl.loop(0, TS // 136)
        def _out(k):
            csl = pl.ds(s * TS + k * 136, 136)
            pltpu.sync_copy(acc.at[csl], bounce)
            pltpu.sync_copy(bounce, out_ref.at[csl])

        plsc.subcore_barrier()

    @pl.when(c == 0)
    def _():
        quarter(z0_hbm, o0_hbm)
        quarter(z1_hbm, o1_hbm)

    @pl.when(c == 1)
    def _():
        quarter(z2_hbm, o2_hbm)
        quarter(z3_hbm, o3_hbm)


def _tc_dinv_y(degp, xp):
    def body(dp_ref, x_ref, dinv_ref, y_ref):
        deg = jnp.sum(dp_ref[...], axis=0, keepdims=True) + 1.0
        dinv = lax.rsqrt(deg)
        dinv_ref[...] = dinv
        y_ref[...] = dinv * x_ref[...]

    return pl.pallas_call(
        body,
        out_shape=(jax.ShapeDtypeStruct((1, NP), F32),
                   jax.ShapeDtypeStruct((1, NP), F32)),
    )(degp, xp)


def _tc_s_stats(aggp, y, dinv):
    def body(ap_ref, y_ref, dinv_ref, s_ref, stats_ref):
        agg = jnp.sum(ap_ref[...], axis=0, keepdims=True) + y_ref[...]
        sv = dinv_ref[...] * agg
        s_ref[...] = sv
        col = lax.broadcasted_iota(jnp.int32, (1, NP), 1)
        msk = (col < N).astype(F32)
        sm = jnp.sum(sv * msk)
        sq = jnp.sum(sv * sv * msk)
        m = sm * (1.0 / N)
        v = sq * (1.0 / N) - m * m
        ri = lax.broadcasted_iota(jnp.int32, (8, 128), 0)
        stats_ref[...] = jnp.where(ri == 0, m, v)

    return pl.pallas_call(
        body,
        out_shape=(jax.ShapeDtypeStruct((1, NP), F32),
                   jax.ShapeDtypeStruct((8, 128), F32)),
    )(aggp, y, dinv)


def _full(shape):
    return pl.BlockSpec(shape, lambda i: (0,) * len(shape))


_QBLK = pl.BlockSpec((BN, 16), lambda i: (i, 0))


def _tc_layer1(s, dinv, stats, w1t, g1t, be1t, w2q):
    def body(s_ref, dinv_ref, stats_ref, w1t_ref, g1t_ref, be1t_ref,
             wq0_ref, wq1_ref, wq2_ref, wq3_ref,
             z0_ref, z1_ref, z2_ref, z3_ref):
        m = stats_ref[0:1, 0:1]
        v = stats_ref[1:2, 0:1]
        w1 = w1t_ref[...]
        a = w1 * g1t_ref[...] * lax.rsqrt(v * w1 * w1 + EPS)
        t = s_ref[...] - m
        x1d = jnp.maximum(a * t + be1t_ref[...], 0.0) * dinv_ref[...]
        dn = (((0,), (0,)), ((), ()))
        z0_ref[...] = lax.dot_general(x1d, wq0_ref[...], dn,
                                      preferred_element_type=F32)
        z1_ref[...] = lax.dot_general(x1d, wq1_ref[...], dn,
                                      preferred_element_type=F32)
        z2_ref[...] = lax.dot_general(x1d, wq2_ref[...], dn,
                                      preferred_element_type=F32)
        z3_ref[...] = lax.dot_general(x1d, wq3_ref[...], dn,
                                      preferred_element_type=F32)

    return pl.pallas_call(
        body,
        grid=(GRID,),
        in_specs=[
            pl.BlockSpec((1, BN), lambda i: (0, i)),
            pl.BlockSpec((1, BN), lambda i: (0, i)),
            _full((8, 128)),
            _full((256, 1)), _full((256, 1)), _full((256, 1)),
            _full((256, 16)), _full((256, 16)), _full((256, 16)),
            _full((256, 16)),
        ],
        out_specs=(_QBLK, _QBLK, _QBLK, _QBLK),
        out_shape=(_QSHAPE, _QSHAPE, _QSHAPE, _QSHAPE),
    )(s, dinv, stats, w1t, g1t, be1t, *w2q)


def _tc_stats2(aggq, dinv16, b2q):
    def body(a0_ref, a1_ref, a2_ref, a3_ref, dinv_ref,
             b0_ref, b1_ref, b2_ref, b3_ref,
             s0_ref, s1_ref, s2_ref, s3_ref):
        i = pl.program_id(0)
        row = lax.broadcasted_iota(jnp.int32, (BN, 16), 0) + i * BN
        msk = (row < N).astype(F32)
        d = dinv_ref[...]
        ri = lax.broadcasted_iota(jnp.int32, (8, 16), 0)
        for a_ref, b_ref, s_ref in ((a0_ref, b0_ref, s0_ref),
                                    (a1_ref, b1_ref, s1_ref),
                                    (a2_ref, b2_ref, s2_ref),
                                    (a3_ref, b3_ref, s3_ref)):
            x = (d * a_ref[...] + b_ref[...]) * msk
            blk = jnp.where(ri == 0, jnp.sum(x, axis=0, keepdims=True),
                            jnp.sum(x * x, axis=0, keepdims=True))

            @pl.when(i == 0)
            def _(s_ref=s_ref, blk=blk):
                s_ref[...] = blk

            @pl.when(i > 0)
            def _(s_ref=s_ref, blk=blk):
                s_ref[...] += blk

    stat_shape = jax.ShapeDtypeStruct((8, 16), F32)
    stat_blk = pl.BlockSpec((8, 16), lambda i: (0, 0))
    return pl.pallas_call(
        body,
        grid=(GRID,),
        in_specs=[_QBLK, _QBLK, _QBLK, _QBLK, _QBLK,
                  _full((1, 16)), _full((1, 16)), _full((1, 16)),
                  _full((1, 16))],
        out_specs=(stat_blk,) * 4,
        out_shape=(stat_shape,) * 4,
    )(*aggq, dinv16, *b2q)


def _tc_layer2(aggq, dinv16, b2q, statq, g2q, be2q, w3q):
    def body(a0_ref, a1_ref, a2_ref, a3_ref, dinv_ref,
             b0_ref, b1_ref, b2_ref, b3_ref,
             s0_ref, s1_ref, s2_ref, s3_ref,
             g0_ref, g1_ref, g2_ref, g3_ref,
             e0_ref, e1_ref, e2_ref, e3_ref,
             w0_ref, w1_ref, w2_ref, w3_ref,
             z0_ref, z1_ref, z2_ref, z3_ref):
        d = dinv_ref[...]
        inv_n = 1.0 / N

        def norm(a_ref, b_ref, s_ref, g_ref, e_ref):
            xpre = d * a_ref[...] + b_ref[...]
            mu = s_ref[0:1, :] * inv_n
            var = s_ref[1:2, :] * inv_n - mu * mu
            return jnp.maximum(
                (xpre - mu) * lax.rsqrt(var + EPS) * g_ref[...] + e_ref[...],
                0.0)

        x2 = jnp.concatenate([
            norm(a0_ref, b0_ref, s0_ref, g0_ref, e0_ref),
            norm(a1_ref, b1_ref, s1_ref, g1_ref, e1_ref),
            norm(a2_ref, b2_ref, s2_ref, g2_ref, e2_ref),
            norm(a3_ref, b3_ref, s3_ref, g3_ref, e3_ref)], axis=1)
        for w_ref, z_ref in ((w0_ref, z0_ref), (w1_ref, z1_ref),
                             (w2_ref, z2_ref), (w3_ref, z3_ref)):
            z_ref[...] = d * jnp.dot(x2, w_ref[...],
                                     preferred_element_type=F32)

    stat_blk = pl.BlockSpec((8, 16), lambda i: (0, 0))
    return pl.pallas_call(
        body,
        grid=(GRID,),
        in_specs=[_QBLK, _QBLK, _QBLK, _QBLK, _QBLK,
                  _full((1, 16)), _full((1, 16)), _full((1, 16)),
                  _full((1, 16)),
                  stat_blk, stat_blk, stat_blk, stat_blk,
                  _full((1, 16)), _full((1, 16)), _full((1, 16)),
                  _full((1, 16)),
                  _full((1, 16)), _full((1, 16)), _full((1, 16)),
                  _full((1, 16)),
                  _full((64, 16)), _full((64, 16)), _full((64, 16)),
                  _full((64, 16))],
        out_specs=(_QBLK, _QBLK, _QBLK, _QBLK),
        out_shape=(_QSHAPE, _QSHAPE, _QSHAPE, _QSHAPE),
    )(*aggq, dinv16, *b2q, *statq, *g2q, *be2q, *w3q)


def _tc_final(aggq, dinv16, b3q):
    def body(a0_ref, a1_ref, a2_ref, a3_ref, dinv_ref,
             b0_ref, b1_ref, b2_ref, b3_ref, o_ref):
        d = dinv_ref[...]
        o_ref[...] = jnp.concatenate(
            [jnp.maximum(d * a_ref[...] + b_ref[...], 0.0)
             for a_ref, b_ref in ((a0_ref, b0_ref), (a1_ref, b1_ref),
                                  (a2_ref, b2_ref), (a3_ref, b3_ref))],
            axis=1)

    return pl.pallas_call(
        body,
        grid=(GRID,),
        in_specs=[_QBLK, _QBLK, _QBLK, _QBLK, _QBLK,
                  _full((1, 16)), _full((1, 16)), _full((1, 16)),
                  _full((1, 16))],
        out_specs=pl.BlockSpec((BN, 64), lambda i: (i, 0)),
        out_shape=jax.ShapeDtypeStruct((N, 64), F32),
    )(*aggq, dinv16, *b3q)


def kernel(data, edge_index, W1, b1, g1, be1, W2, b2, g2, be2, W3, b3):
    padn = EP - E
    fill_src = lax.iota(jnp.int32, padn) % 128
    fill_dst = N + lax.iota(jnp.int32, padn) % (NP - N)
    src2d = jnp.concatenate([edge_index[0], fill_src]).reshape(ERP, 128)
    dst2d = jnp.concatenate([edge_index[1], fill_dst]).reshape(ERP, 128)
    xp = jnp.pad(data[:, 0], (0, NP - N)).reshape(1, NP)

    w1t = W1.reshape(256, 1)
    g1t = g1.reshape(256, 1)
    be1t = be1.reshape(256, 1)
    quarters = lambda v: [v[..., 16 * q:16 * (q + 1)] for q in range(4)]
    w2q = quarters(W2)
    b2q = [b.reshape(1, 16) for b in quarters(b2)]
    g2q = [b.reshape(1, 16) for b in quarters(g2)]
    be2q = [b.reshape(1, 16) for b in quarters(be2)]
    w3q = quarters(W3)
    b3q = [b.reshape(1, 16) for b in quarters(b3)]

    degp = _sc_deg(dst2d)
    dinv, y = _tc_dinv_y(degp.reshape(32, NP), xp)
    aggp = _sc_agg_scalar(src2d, dst2d, y.reshape(NP))
    s, stats = _tc_s_stats(aggp.reshape(32, NP), y, dinv)
    z2q = _tc_layer1(s, dinv, stats, w1t, g1t, be1t, w2q)
    dinv16 = jnp.broadcast_to(dinv.reshape(NP, 1), (NP, 16))
    a2q = _sc_agg_wide(*z2q, src2d, dst2d)
    statq = _tc_stats2(a2q, dinv16, b2q)
    z3q = _tc_layer2(a2q, dinv16, b2q, statq, g2q, be2q, w3q)
    a3q = _sc_agg_wide(*z3q, src2d, dst2d)
    return _tc_final(a3q, dinv16, b3q)


# speedup vs baseline: 24.6764x; 1.6685x over previous
import functools

import jax
import jax.numpy as jnp
from jax import lax
from jax.experimental import pallas as pl
from jax.experimental.pallas import tpu as pltpu
from jax.experimental.pallas import tpu_sc as plsc

N = 50000
NP = 50048
E = 800000
ERP = 6400
EP = ERP * 128
RT2 = ERP // 32
RT1 = ERP // 16
EPS = 1e-5
BN = 2944
GRID = NP // BN
TS = NP // 16
F32 = jnp.float32

_MESH = plsc.VectorSubcoreMesh(
    core_axis_name="c", subcore_axis_name="s", num_cores=2, num_subcores=16)


@functools.partial(
    pl.kernel,
    out_type=jax.ShapeDtypeStruct((32 * NP,), F32),
    mesh=_MESH,
    scratch_types=[
        pltpu.VMEM((RT2, 128), jnp.int32),
        pltpu.VMEM((NP,), F32),
    ],
    compiler_params=pltpu.CompilerParams(needs_layout_passes=False),
)
def _sc_deg(dst_hbm, out_hbm, dstv, acc):
    c = lax.axis_index("c")
    s = lax.axis_index("s")
    w = c * 16 + s

    @pl.loop(0, NP // 16)
    def _zero(i):
        acc[pl.ds(i * 16, 16)] = jnp.zeros((16,), F32)

    pltpu.sync_copy(dst_hbm.at[pl.ds(w * RT2, RT2)], dstv)
    ones = jnp.ones((16,), F32)

    @pl.loop(0, RT2)
    def _edges(j):
        for b in range(8):
            plsc.addupdate_scatter(acc, [dstv[j, pl.ds(b * 16, 16)]], ones)

    pltpu.sync_copy(acc, out_hbm.at[pl.ds(w * NP, NP)])


@functools.partial(
    pl.kernel,
    out_type=jax.ShapeDtypeStruct((32 * NP,), F32),
    mesh=_MESH,
    scratch_types=[
        pltpu.VMEM((8, 128), jnp.int32),
        pltpu.VMEM((8, 128), jnp.int32),
        pltpu.VMEM((NP,), F32),
        pltpu.VMEM((NP,), F32),
    ],
    compiler_params=pltpu.CompilerParams(needs_layout_passes=False),
)
def _sc_agg_scalar(src_hbm, dst_hbm, y_hbm, out_hbm, srcv, dstv, yv, acc):
    c = lax.axis_index("c")
    s = lax.axis_index("s")
    w = c * 16 + s

    @pl.loop(0, NP // 16)
    def _zero(i):
        acc[pl.ds(i * 16, 16)] = jnp.zeros((16,), F32)

    pltpu.sync_copy(y_hbm, yv)

    @pl.loop(0, RT2 // 8)
    def _chunk(k):
        rsl = pl.ds(w * RT2 + k * 8, 8)
        pltpu.sync_copy(src_hbm.at[rsl], srcv)
        pltpu.sync_copy(dst_hbm.at[rsl], dstv)

        @pl.loop(0, 8)
        def _row(j):
            for b in range(8):
                lane = pl.ds(b * 16, 16)
                vals = plsc.load_gather(yv, [srcv[j, lane]])
                plsc.addupdate_scatter(acc, [dstv[j, lane]], vals)

    pltpu.sync_copy(acc, out_hbm.at[pl.ds(w * NP, NP)])


_QSHAPE = jax.ShapeDtypeStruct((NP, 16), F32)


@functools.partial(
    pl.kernel,
    out_type=(_QSHAPE, _QSHAPE, _QSHAPE, _QSHAPE),
    mesh=_MESH,
    scratch_types=[
        pltpu.VMEM((5120,), jnp.int32),
        pltpu.VMEM((40, 128), jnp.int32),
        pltpu.VMEM((1024, 16), F32),
        pltpu.VMEM((136, 16), F32),
        pltpu.SemaphoreType.DMA,
        pltpu.SemaphoreType.DMA,
        pltpu.VMEM_SHARED((NP, 16), F32),
    ],
    compiler_params=pltpu.CompilerParams(use_tc_tiling_on_sc=False),
)
def _sc_agg_wide(z0_hbm, z1_hbm, z2_hbm, z3_hbm, srcf_hbm, dst_hbm,
                 o0_hbm, o1_hbm, o2_hbm, o3_hbm,
                 srcv, dstv, rowv, bounce, gsem, ssem, acc):
    c = lax.axis_index("c")
    s = lax.axis_index("s")

    def quarter(z_ref, out_ref):
        @pl.loop(0, TS // 136)
        def _init(k):
            csl = pl.ds(s * TS + k * 136, 136)
            pltpu.sync_copy(z_ref.at[csl], bounce)
            pltpu.sync_copy(bounce, acc.at[csl])

        plsc.subcore_barrier()

        @pl.loop(0, RT1 // 40)
        def _sup(k):
            row0 = s * RT1 + k * 40
            pltpu.sync_copy(srcf_hbm.at[pl.ds(row0 * 128, 5120)], srcv)
            pltpu.sync_copy(dst_hbm.at[pl.ds(row0, 40)], dstv)
            for t in range(5):
                g = pltpu.async_copy(
                    z_ref.at[srcv.at[pl.ds(t * 1024, 1024)]], rowv, gsem)
                g.wait()
                scat = [
                    pltpu.async_copy(
                        rowv.at[pl.ds(b * 128, 128)],
                        acc.at[dstv.at[t * 8 + b]], ssem, add=True)
                    for b in range(8)
                ]
                for d in scat:
                    d.wait()

        plsc.subcore_barrier()

        @pl.loop(0, TS // 136)
        def _out(k):
            csl = pl.ds(s * TS + k * 136, 136)
            pltpu.sync_copy(acc.at[csl], bounce)
            pltpu.sync_copy(bounce, out_ref.at[csl])

        plsc.subcore_barrier()

    @pl.when(c == 0)
    def _():
        quarter(z0_hbm, o0_hbm)
        quarter(z1_hbm, o1_hbm)

    @pl.when(c == 1)
    def _():
        quarter(z2_hbm, o2_hbm)
        quarter(z3_hbm, o3_hbm)


def _tc_dinv_y(degp, xp):
    def body(dp_ref, x_ref, dinv_ref, y_ref):
        deg = jnp.sum(dp_ref[...], axis=0, keepdims=True) + 1.0
        dinv = lax.rsqrt(deg)
        dinv_ref[...] = dinv
        y_ref[...] = dinv * x_ref[...]

    return pl.pallas_call(
        body,
        out_shape=(jax.ShapeDtypeStruct((1, NP), F32),
                   jax.ShapeDtypeStruct((1, NP), F32)),
    )(degp, xp)


def _tc_s_stats(aggp, y, dinv):
    def body(ap_ref, y_ref, dinv_ref, s_ref, stats_ref):
        agg = jnp.sum(ap_ref[...], axis=0, keepdims=True) + y_ref[...]
        sv = dinv_ref[...] * agg
        s_ref[...] = sv
        col = lax.broadcasted_iota(jnp.int32, (1, NP), 1)
        msk = (col < N).astype(F32)
        sm = jnp.sum(sv * msk)
        sq = jnp.sum(sv * sv * msk)
        m = sm * (1.0 / N)
        v = sq * (1.0 / N) - m * m
        ri = lax.broadcasted_iota(jnp.int32, (8, 128), 0)
        stats_ref[...] = jnp.where(ri == 0, m, v)

    return pl.pallas_call(
        body,
        out_shape=(jax.ShapeDtypeStruct((1, NP), F32),
                   jax.ShapeDtypeStruct((8, 128), F32)),
    )(aggp, y, dinv)


def _full(shape):
    return pl.BlockSpec(shape, lambda i: (0,) * len(shape))


_QBLK = pl.BlockSpec((BN, 16), lambda i: (i, 0))


def _tc_layer1(s, dinv, stats, w1t, g1t, be1t, w2q):
    def body(s_ref, dinv_ref, stats_ref, w1t_ref, g1t_ref, be1t_ref,
             wq0_ref, wq1_ref, wq2_ref, wq3_ref,
             z0_ref, z1_ref, z2_ref, z3_ref):
        m = stats_ref[0:1, 0:1]
        v = stats_ref[1:2, 0:1]
        w1 = w1t_ref[...]
        a = w1 * g1t_ref[...] * lax.rsqrt(v * w1 * w1 + EPS)
        t = s_ref[...] - m
        x1d = jnp.maximum(a * t + be1t_ref[...], 0.0) * dinv_ref[...]
        dn = (((0,), (0,)), ((), ()))
        z0_ref[...] = lax.dot_general(x1d, wq0_ref[...], dn,
                                      preferred_element_type=F32)
        z1_ref[...] = lax.dot_general(x1d, wq1_ref[...], dn,
                                      preferred_element_type=F32)
        z2_ref[...] = lax.dot_general(x1d, wq2_ref[...], dn,
                                      preferred_element_type=F32)
        z3_ref[...] = lax.dot_general(x1d, wq3_ref[...], dn,
                                      preferred_element_type=F32)

    return pl.pallas_call(
        body,
        grid=(GRID,),
        in_specs=[
            pl.BlockSpec((1, BN), lambda i: (0, i)),
            pl.BlockSpec((1, BN), lambda i: (0, i)),
            _full((8, 128)),
            _full((256, 1)), _full((256, 1)), _full((256, 1)),
            _full((256, 16)), _full((256, 16)), _full((256, 16)),
            _full((256, 16)),
        ],
        out_specs=(_QBLK, _QBLK, _QBLK, _QBLK),
        out_shape=(_QSHAPE, _QSHAPE, _QSHAPE, _QSHAPE),
    )(s, dinv, stats, w1t, g1t, be1t, *w2q)


def _tc_stats2(aggq, dinv16, b2q):
    def body(a0_ref, a1_ref, a2_ref, a3_ref, dinv_ref,
             b0_ref, b1_ref, b2_ref, b3_ref,
             s0_ref, s1_ref, s2_ref, s3_ref):
        i = pl.program_id(0)
        row = lax.broadcasted_iota(jnp.int32, (BN, 16), 0) + i * BN
        msk = (row < N).astype(F32)
        d = dinv_ref[...]
        ri = lax.broadcasted_iota(jnp.int32, (8, 16), 0)
        for a_ref, b_ref, s_ref in ((a0_ref, b0_ref, s0_ref),
                                    (a1_ref, b1_ref, s1_ref),
                                    (a2_ref, b2_ref, s2_ref),
                                    (a3_ref, b3_ref, s3_ref)):
            x = (d * a_ref[...] + b_ref[...]) * msk
            blk = jnp.where(ri == 0, jnp.sum(x, axis=0, keepdims=True),
                            jnp.sum(x * x, axis=0, keepdims=True))

            @pl.when(i == 0)
            def _(s_ref=s_ref, blk=blk):
                s_ref[...] = blk

            @pl.when(i > 0)
            def _(s_ref=s_ref, blk=blk):
                s_ref[...] += blk

    stat_shape = jax.ShapeDtypeStruct((8, 16), F32)
    stat_blk = pl.BlockSpec((8, 16), lambda i: (0, 0))
    return pl.pallas_call(
        body,
        grid=(GRID,),
        in_specs=[_QBLK, _QBLK, _QBLK, _QBLK, _QBLK,
                  _full((1, 16)), _full((1, 16)), _full((1, 16)),
                  _full((1, 16))],
        out_specs=(stat_blk,) * 4,
        out_shape=(stat_shape,) * 4,
    )(*aggq, dinv16, *b2q)


def _tc_layer2(aggq, dinv16, b2q, statq, g2q, be2q, w3q):
    def body(a0_ref, a1_ref, a2_ref, a3_ref, dinv_ref,
             b0_ref, b1_ref, b2_ref, b3_ref,
             s0_ref, s1_ref, s2_ref, s3_ref,
             g0_ref, g1_ref, g2_ref, g3_ref,
             e0_ref, e1_ref, e2_ref, e3_ref,
             w0_ref, w1_ref, w2_ref, w3_ref,
             z0_ref, z1_ref, z2_ref, z3_ref):
        d = dinv_ref[...]
        inv_n = 1.0 / N

        def norm(a_ref, b_ref, s_ref, g_ref, e_ref):
            xpre = d * a_ref[...] + b_ref[...]
            mu = s_ref[0:1, :] * inv_n
            var = s_ref[1:2, :] * inv_n - mu * mu
            return jnp.maximum(
                (xpre - mu) * lax.rsqrt(var + EPS) * g_ref[...] + e_ref[...],
                0.0)

        x2 = jnp.concatenate([
            norm(a0_ref, b0_ref, s0_ref, g0_ref, e0_ref),
            norm(a1_ref, b1_ref, s1_ref, g1_ref, e1_ref),
            norm(a2_ref, b2_ref, s2_ref, g2_ref, e2_ref),
            norm(a3_ref, b3_ref, s3_ref, g3_ref, e3_ref)], axis=1)
        for w_ref, z_ref in ((w0_ref, z0_ref), (w1_ref, z1_ref),
                             (w2_ref, z2_ref), (w3_ref, z3_ref)):
            z_ref[...] = d * jnp.dot(x2, w_ref[...],
                                     preferred_element_type=F32)

    stat_blk = pl.BlockSpec((8, 16), lambda i: (0, 0))
    return pl.pallas_call(
        body,
        grid=(GRID,),
        in_specs=[_QBLK, _QBLK, _QBLK, _QBLK, _QBLK,
                  _full((1, 16)), _full((1, 16)), _full((1, 16)),
                  _full((1, 16)),
                  stat_blk, stat_blk, stat_blk, stat_blk,
                  _full((1, 16)), _full((1, 16)), _full((1, 16)),
                  _full((1, 16)),
                  _full((1, 16)), _full((1, 16)), _full((1, 16)),
                  _full((1, 16)),
                  _full((64, 16)), _full((64, 16)), _full((64, 16)),
                  _full((64, 16))],
        out_specs=(_QBLK, _QBLK, _QBLK, _QBLK),
        out_shape=(_QSHAPE, _QSHAPE, _QSHAPE, _QSHAPE),
    )(*aggq, dinv16, *b2q, *statq, *g2q, *be2q, *w3q)


def _tc_final(aggq, dinv16, b3q):
    def body(a0_ref, a1_ref, a2_ref, a3_ref, dinv_ref,
             b0_ref, b1_ref, b2_ref, b3_ref, o_ref):
        d = dinv_ref[...]
        o_ref[...] = jnp.concatenate(
            [jnp.maximum(d * a_ref[...] + b_ref[...], 0.0)
             for a_ref, b_ref in ((a0_ref, b0_ref), (a1_ref, b1_ref),
                                  (a2_ref, b2_ref), (a3_ref, b3_ref))],
            axis=1)

    return pl.pallas_call(
        body,
        grid=(GRID,),
        in_specs=[_QBLK, _QBLK, _QBLK, _QBLK, _QBLK,
                  _full((1, 16)), _full((1, 16)), _full((1, 16)),
                  _full((1, 16))],
        out_specs=pl.BlockSpec((BN, 64), lambda i: (i, 0)),
        out_shape=jax.ShapeDtypeStruct((N, 64), F32),
    )(*aggq, dinv16, *b3q)


def kernel(data, edge_index, W1, b1, g1, be1, W2, b2, g2, be2, W3, b3):
    padn = EP - E
    fill_src = lax.iota(jnp.int32, padn) % 128
    fill_dst = N + lax.iota(jnp.int32, padn) % (NP - N)
    src2d = jnp.concatenate([edge_index[0], fill_src]).reshape(ERP, 128)
    dst2d = jnp.concatenate([edge_index[1], fill_dst]).reshape(ERP, 128)
    xp = jnp.pad(data[:, 0], (0, NP - N)).reshape(1, NP)

    w1t = W1.reshape(256, 1)
    g1t = g1.reshape(256, 1)
    be1t = be1.reshape(256, 1)
    quarters = lambda v: [v[..., 16 * q:16 * (q + 1)] for q in range(4)]
    w2q = quarters(W2)
    b2q = [b.reshape(1, 16) for b in quarters(b2)]
    g2q = [b.reshape(1, 16) for b in quarters(g2)]
    be2q = [b.reshape(1, 16) for b in quarters(be2)]
    w3q = quarters(W3)
    b3q = [b.reshape(1, 16) for b in quarters(b3)]

    degp = _sc_deg(dst2d)
    dinv, y = _tc_dinv_y(degp.reshape(32, NP), xp)
    aggp = _sc_agg_scalar(src2d, dst2d, y.reshape(NP))
    s, stats = _tc_s_stats(aggp.reshape(32, NP), y, dinv)
    z2q = _tc_layer1(s, dinv, stats, w1t, g1t, be1t, w2q)
    dinv16 = jnp.broadcast_to(dinv.reshape(NP, 1), (NP, 16))
    srcf = src2d.reshape(EP)
    a2q = _sc_agg_wide(*z2q, srcf, dst2d)
    statq = _tc_stats2(a2q, dinv16, b2q)
    z3q = _tc_layer2(a2q, dinv16, b2q, statq, g2q, be2q, w3q)
    a3q = _sc_agg_wide(*z3q, srcf, dst2d)
    return _tc_final(a3q, dinv16, b3q)


# speedup vs baseline: 26.6397x; 1.0796x over previous
import functools

import jax
import jax.numpy as jnp
from jax import lax
from jax.experimental import pallas as pl
from jax.experimental.pallas import tpu as pltpu
from jax.experimental.pallas import tpu_sc as plsc

N = 50000
NP = 50048
E = 800000
ERP = 6400
EP = ERP * 128
RT2 = ERP // 32
RT1 = ERP // 16
EPS = 1e-5
BN = 2944
GRID = NP // BN
TS = NP // 16
F32 = jnp.float32

_MESH = plsc.VectorSubcoreMesh(
    core_axis_name="c", subcore_axis_name="s", num_cores=2, num_subcores=16)


@functools.partial(
    pl.kernel,
    out_type=jax.ShapeDtypeStruct((32 * NP,), F32),
    mesh=_MESH,
    scratch_types=[
        pltpu.VMEM((RT2, 128), jnp.int32),
        pltpu.VMEM((NP,), F32),
    ],
    compiler_params=pltpu.CompilerParams(needs_layout_passes=False),
)
def _sc_deg(dst_hbm, out_hbm, dstv, acc):
    c = lax.axis_index("c")
    s = lax.axis_index("s")
    w = c * 16 + s

    @pl.loop(0, NP // 16)
    def _zero(i):
        acc[pl.ds(i * 16, 16)] = jnp.zeros((16,), F32)

    pltpu.sync_copy(dst_hbm.at[pl.ds(w * RT2, RT2)], dstv)
    ones = jnp.ones((16,), F32)

    @pl.loop(0, RT2)
    def _edges(j):
        for b in range(8):
            plsc.addupdate_scatter(acc, [dstv[j, pl.ds(b * 16, 16)]], ones)

    pltpu.sync_copy(acc, out_hbm.at[pl.ds(w * NP, NP)])


@functools.partial(
    pl.kernel,
    out_type=jax.ShapeDtypeStruct((32 * NP,), F32),
    mesh=_MESH,
    scratch_types=[
        pltpu.VMEM((8, 128), jnp.int32),
        pltpu.VMEM((8, 128), jnp.int32),
        pltpu.VMEM((NP,), F32),
        pltpu.VMEM((NP,), F32),
    ],
    compiler_params=pltpu.CompilerParams(needs_layout_passes=False),
)
def _sc_agg_scalar(src_hbm, dst_hbm, y_hbm, out_hbm, srcv, dstv, yv, acc):
    c = lax.axis_index("c")
    s = lax.axis_index("s")
    w = c * 16 + s

    @pl.loop(0, NP // 16)
    def _zero(i):
        acc[pl.ds(i * 16, 16)] = jnp.zeros((16,), F32)

    pltpu.sync_copy(y_hbm, yv)

    @pl.loop(0, RT2 // 8)
    def _chunk(k):
        rsl = pl.ds(w * RT2 + k * 8, 8)
        pltpu.sync_copy(src_hbm.at[rsl], srcv)
        pltpu.sync_copy(dst_hbm.at[rsl], dstv)

        @pl.loop(0, 8)
        def _row(j):
            for b in range(8):
                lane = pl.ds(b * 16, 16)
                vals = plsc.load_gather(yv, [srcv[j, lane]])
                plsc.addupdate_scatter(acc, [dstv[j, lane]], vals)

    pltpu.sync_copy(acc, out_hbm.at[pl.ds(w * NP, NP)])


_QSHAPE = jax.ShapeDtypeStruct((NP, 16), F32)


@functools.partial(
    pl.kernel,
    out_type=(_QSHAPE, _QSHAPE, _QSHAPE, _QSHAPE),
    mesh=_MESH,
    scratch_types=[
        pltpu.VMEM((5120,), jnp.int32),
        pltpu.VMEM((40, 128), jnp.int32),
        pltpu.VMEM((1024, 16), F32),
        pltpu.VMEM((1024, 16), F32),
        pltpu.VMEM((136, 16), F32),
        pltpu.SemaphoreType.DMA,
        pltpu.SemaphoreType.DMA,
        pltpu.VMEM_SHARED((NP, 16), F32),
    ],
    compiler_params=pltpu.CompilerParams(use_tc_tiling_on_sc=False),
)
def _sc_agg_wide(z0_hbm, z1_hbm, z2_hbm, z3_hbm, srcf_hbm, dst_hbm,
                 o0_hbm, o1_hbm, o2_hbm, o3_hbm,
                 srcv, dstv, rowv0, rowv1, bounce, gsem, ssem, acc):
    c = lax.axis_index("c")
    s = lax.axis_index("s")

    def quarter(z_ref, out_ref):
        @pl.loop(0, TS // 136)
        def _init(k):
            csl = pl.ds(s * TS + k * 136, 136)
            pltpu.sync_copy(z_ref.at[csl], bounce)
            pltpu.sync_copy(bounce, acc.at[csl])

        plsc.subcore_barrier()

        @pl.loop(0, RT1 // 40)
        def _sup(k):
            row0 = s * RT1 + k * 40
            pltpu.sync_copy(srcf_hbm.at[pl.ds(row0 * 128, 5120)], srcv)
            pltpu.sync_copy(dst_hbm.at[pl.ds(row0, 40)], dstv)
            bufs = (rowv0, rowv1)

            def fire_gather(t):
                return pltpu.async_copy(
                    z_ref.at[srcv.at[pl.ds(t * 1024, 1024)]],
                    bufs[t % 2], gsem)

            def fire_scatters(t):
                return [
                    pltpu.async_copy(
                        bufs[t % 2].at[pl.ds(b * 128, 128)],
                        acc.at[dstv.at[t * 8 + b]], ssem, add=True)
                    for b in range(8)
                ]

            g = fire_gather(0)
            pend = []
            for t in range(5):
                if t + 1 < 5:
                    if t >= 1:
                        for d in pend:
                            d.wait()
                    g.wait()
                    gn = fire_gather(t + 1)
                else:
                    g.wait()
                    gn = None
                pend_new = fire_scatters(t)
                if t + 1 == 5:
                    for d in pend:
                        d.wait()
                pend = pend_new
                g = gn
            for d in pend:
                d.wait()

        plsc.subcore_barrier()

        @pl.loop(0, TS // 136)
        def _out(k):
            csl = pl.ds(s * TS + k * 136, 136)
            pltpu.sync_copy(acc.at[csl], bounce)
            pltpu.sync_copy(bounce, out_ref.at[csl])

        plsc.subcore_barrier()

    @pl.when(c == 0)
    def _():
        quarter(z0_hbm, o0_hbm)
        quarter(z1_hbm, o1_hbm)

    @pl.when(c == 1)
    def _():
        quarter(z2_hbm, o2_hbm)
        quarter(z3_hbm, o3_hbm)


def _tc_dinv_y(degp, xp):
    def body(dp_ref, x_ref, dinv_ref, y_ref):
        deg = jnp.sum(dp_ref[...], axis=0, keepdims=True) + 1.0
        dinv = lax.rsqrt(deg)
        dinv_ref[...] = dinv
        y_ref[...] = dinv * x_ref[...]

    return pl.pallas_call(
        body,
        out_shape=(jax.ShapeDtypeStruct((1, NP), F32),
                   jax.ShapeDtypeStruct((1, NP), F32)),
    )(degp, xp)


def _tc_s_stats(aggp, y, dinv):
    def body(ap_ref, y_ref, dinv_ref, s_ref, stats_ref):
        agg = jnp.sum(ap_ref[...], axis=0, keepdims=True) + y_ref[...]
        sv = dinv_ref[...] * agg
        s_ref[...] = sv
        col = lax.broadcasted_iota(jnp.int32, (1, NP), 1)
        msk = (col < N).astype(F32)
        sm = jnp.sum(sv * msk)
        sq = jnp.sum(sv * sv * msk)
        m = sm * (1.0 / N)
        v = sq * (1.0 / N) - m * m
        ri = lax.broadcasted_iota(jnp.int32, (8, 128), 0)
        stats_ref[...] = jnp.where(ri == 0, m, v)

    return pl.pallas_call(
        body,
        out_shape=(jax.ShapeDtypeStruct((1, NP), F32),
                   jax.ShapeDtypeStruct((8, 128), F32)),
    )(aggp, y, dinv)


def _full(shape):
    return pl.BlockSpec(shape, lambda i: (0,) * len(shape))


_QBLK = pl.BlockSpec((BN, 16), lambda i: (i, 0))


def _tc_layer1(s, dinv, stats, w1t, g1t, be1t, w2q):
    def body(s_ref, dinv_ref, stats_ref, w1t_ref, g1t_ref, be1t_ref,
             wq0_ref, wq1_ref, wq2_ref, wq3_ref,
             z0_ref, z1_ref, z2_ref, z3_ref):
        m = stats_ref[0:1, 0:1]
        v = stats_ref[1:2, 0:1]
        w1 = w1t_ref[...]
        a = w1 * g1t_ref[...] * lax.rsqrt(v * w1 * w1 + EPS)
        t = s_ref[...] - m
        x1d = jnp.maximum(a * t + be1t_ref[...], 0.0) * dinv_ref[...]
        dn = (((0,), (0,)), ((), ()))
        z0_ref[...] = lax.dot_general(x1d, wq0_ref[...], dn,
                                      preferred_element_type=F32)
        z1_ref[...] = lax.dot_general(x1d, wq1_ref[...], dn,
                                      preferred_element_type=F32)
        z2_ref[...] = lax.dot_general(x1d, wq2_ref[...], dn,
                                      preferred_element_type=F32)
        z3_ref[...] = lax.dot_general(x1d, wq3_ref[...], dn,
                                      preferred_element_type=F32)

    return pl.pallas_call(
        body,
        grid=(GRID,),
        in_specs=[
            pl.BlockSpec((1, BN), lambda i: (0, i)),
            pl.BlockSpec((1, BN), lambda i: (0, i)),
            _full((8, 128)),
            _full((256, 1)), _full((256, 1)), _full((256, 1)),
            _full((256, 16)), _full((256, 16)), _full((256, 16)),
            _full((256, 16)),
        ],
        out_specs=(_QBLK, _QBLK, _QBLK, _QBLK),
        out_shape=(_QSHAPE, _QSHAPE, _QSHAPE, _QSHAPE),
    )(s, dinv, stats, w1t, g1t, be1t, *w2q)


def _tc_stats2(aggq, dinv16, b2q):
    def body(a0_ref, a1_ref, a2_ref, a3_ref, dinv_ref,
             b0_ref, b1_ref, b2_ref, b3_ref,
             s0_ref, s1_ref, s2_ref, s3_ref):
        i = pl.program_id(0)
        row = lax.broadcasted_iota(jnp.int32, (BN, 16), 0) + i * BN
        msk = (row < N).astype(F32)
        d = dinv_ref[...]
        ri = lax.broadcasted_iota(jnp.int32, (8, 16), 0)
        for a_ref, b_ref, s_ref in ((a0_ref, b0_ref, s0_ref),
                                    (a1_ref, b1_ref, s1_ref),
                                    (a2_ref, b2_ref, s2_ref),
                                    (a3_ref, b3_ref, s3_ref)):
            x = (d * a_ref[...] + b_ref[...]) * msk
            blk = jnp.where(ri == 0, jnp.sum(x, axis=0, keepdims=True),
                            jnp.sum(x * x, axis=0, keepdims=True))

            @pl.when(i == 0)
            def _(s_ref=s_ref, blk=blk):
                s_ref[...] = blk

            @pl.when(i > 0)
            def _(s_ref=s_ref, blk=blk):
                s_ref[...] += blk

    stat_shape = jax.ShapeDtypeStruct((8, 16), F32)
    stat_blk = pl.BlockSpec((8, 16), lambda i: (0, 0))
    return pl.pallas_call(
        body,
        grid=(GRID,),
        in_specs=[_QBLK, _QBLK, _QBLK, _QBLK, _QBLK,
                  _full((1, 16)), _full((1, 16)), _full((1, 16)),
                  _full((1, 16))],
        out_specs=(stat_blk,) * 4,
        out_shape=(stat_shape,) * 4,
    )(*aggq, dinv16, *b2q)


def _tc_layer2(aggq, dinv16, b2q, statq, g2q, be2q, w3q):
    def body(a0_ref, a1_ref, a2_ref, a3_ref, dinv_ref,
             b0_ref, b1_ref, b2_ref, b3_ref,
             s0_ref, s1_ref, s2_ref, s3_ref,
             g0_ref, g1_ref, g2_ref, g3_ref,
             e0_ref, e1_ref, e2_ref, e3_ref,
             w0_ref, w1_ref, w2_ref, w3_ref,
             z0_ref, z1_ref, z2_ref, z3_ref):
        d = dinv_ref[...]
        inv_n = 1.0 / N

        def norm(a_ref, b_ref, s_ref, g_ref, e_ref):
            xpre = d * a_ref[...] + b_ref[...]
            mu = s_ref[0:1, :] * inv_n
            var = s_ref[1:2, :] * inv_n - mu * mu
            return jnp.maximum(
                (xpre - mu) * lax.rsqrt(var + EPS) * g_ref[...] + e_ref[...],
                0.0)

        x2 = jnp.concatenate([
            norm(a0_ref, b0_ref, s0_ref, g0_ref, e0_ref),
            norm(a1_ref, b1_ref, s1_ref, g1_ref, e1_ref),
            norm(a2_ref, b2_ref, s2_ref, g2_ref, e2_ref),
            norm(a3_ref, b3_ref, s3_ref, g3_ref, e3_ref)], axis=1)
        for w_ref, z_ref in ((w0_ref, z0_ref), (w1_ref, z1_ref),
                             (w2_ref, z2_ref), (w3_ref, z3_ref)):
            z_ref[...] = d * jnp.dot(x2, w_ref[...],
                                     preferred_element_type=F32)

    stat_blk = pl.BlockSpec((8, 16), lambda i: (0, 0))
    return pl.pallas_call(
        body,
        grid=(GRID,),
        in_specs=[_QBLK, _QBLK, _QBLK, _QBLK, _QBLK,
                  _full((1, 16)), _full((1, 16)), _full((1, 16)),
                  _full((1, 16)),
                  stat_blk, stat_blk, stat_blk, stat_blk,
                  _full((1, 16)), _full((1, 16)), _full((1, 16)),
                  _full((1, 16)),
                  _full((1, 16)), _full((1, 16)), _full((1, 16)),
                  _full((1, 16)),
                  _full((64, 16)), _full((64, 16)), _full((64, 16)),
                  _full((64, 16))],
        out_specs=(_QBLK, _QBLK, _QBLK, _QBLK),
        out_shape=(_QSHAPE, _QSHAPE, _QSHAPE, _QSHAPE),
    )(*aggq, dinv16, *b2q, *statq, *g2q, *be2q, *w3q)


def _tc_final(aggq, dinv16, b3q):
    def body(a0_ref, a1_ref, a2_ref, a3_ref, dinv_ref,
             b0_ref, b1_ref, b2_ref, b3_ref, o_ref):
        d = dinv_ref[...]
        o_ref[...] = jnp.concatenate(
            [jnp.maximum(d * a_ref[...] + b_ref[...], 0.0)
             for a_ref, b_ref in ((a0_ref, b0_ref), (a1_ref, b1_ref),
                                  (a2_ref, b2_ref), (a3_ref, b3_ref))],
            axis=1)

    return pl.pallas_call(
        body,
        grid=(GRID,),
        in_specs=[_QBLK, _QBLK, _QBLK, _QBLK, _QBLK,
                  _full((1, 16)), _full((1, 16)), _full((1, 16)),
                  _full((1, 16))],
        out_specs=pl.BlockSpec((BN, 64), lambda i: (i, 0)),
        out_shape=jax.ShapeDtypeStruct((N, 64), F32),
    )(*aggq, dinv16, *b3q)


def kernel(data, edge_index, W1, b1, g1, be1, W2, b2, g2, be2, W3, b3):
    padn = EP - E
    fill_src = lax.iota(jnp.int32, padn) % 128
    fill_dst = N + lax.iota(jnp.int32, padn) % (NP - N)
    src2d = jnp.concatenate([edge_index[0], fill_src]).reshape(ERP, 128)
    dst2d = jnp.concatenate([edge_index[1], fill_dst]).reshape(ERP, 128)
    xp = jnp.pad(data[:, 0], (0, NP - N)).reshape(1, NP)

    w1t = W1.reshape(256, 1)
    g1t = g1.reshape(256, 1)
    be1t = be1.reshape(256, 1)
    quarters = lambda v: [v[..., 16 * q:16 * (q + 1)] for q in range(4)]
    w2q = quarters(W2)
    b2q = [b.reshape(1, 16) for b in quarters(b2)]
    g2q = [b.reshape(1, 16) for b in quarters(g2)]
    be2q = [b.reshape(1, 16) for b in quarters(be2)]
    w3q = quarters(W3)
    b3q = [b.reshape(1, 16) for b in quarters(b3)]

    degp = _sc_deg(dst2d)
    dinv, y = _tc_dinv_y(degp.reshape(32, NP), xp)
    aggp = _sc_agg_scalar(src2d, dst2d, y.reshape(NP))
    s, stats = _tc_s_stats(aggp.reshape(32, NP), y, dinv)
    z2q = _tc_layer1(s, dinv, stats, w1t, g1t, be1t, w2q)
    dinv16 = jnp.broadcast_to(dinv.reshape(NP, 1), (NP, 16))
    srcf = src2d.reshape(EP)
    a2q = _sc_agg_wide(*z2q, srcf, dst2d)
    statq = _tc_stats2(a2q, dinv16, b2q)
    z3q = _tc_layer2(a2q, dinv16, b2q, statq, g2q, be2q, w3q)
    a3q = _sc_agg_wide(*z3q, srcf, dst2d)
    return _tc_final(a3q, dinv16, b3q)


# speedup vs baseline: 26.8004x; 1.0060x over previous
import functools

import jax
import jax.numpy as jnp
from jax import lax
from jax.experimental import pallas as pl
from jax.experimental.pallas import tpu as pltpu
from jax.experimental.pallas import tpu_sc as plsc

N = 50000
NP = 50048
E = 800000
ERP = 6400
EP = ERP * 128
RT2 = ERP // 32
RT1 = ERP // 16
EPS = 1e-5
BN = 2944
GRID = NP // BN
TS = NP // 16
F32 = jnp.float32

_MESH = plsc.VectorSubcoreMesh(
    core_axis_name="c", subcore_axis_name="s", num_cores=2, num_subcores=16)


@functools.partial(
    pl.kernel,
    out_type=jax.ShapeDtypeStruct((32 * NP,), F32),
    mesh=_MESH,
    scratch_types=[
        pltpu.VMEM((RT2, 128), jnp.int32),
        pltpu.VMEM((NP,), F32),
    ],
    compiler_params=pltpu.CompilerParams(needs_layout_passes=False),
)
def _sc_deg(dst_hbm, out_hbm, dstv, acc):
    c = lax.axis_index("c")
    s = lax.axis_index("s")
    w = c * 16 + s

    @pl.loop(0, NP // 16)
    def _zero(i):
        acc[pl.ds(i * 16, 16)] = jnp.zeros((16,), F32)

    pltpu.sync_copy(dst_hbm.at[pl.ds(w * RT2, RT2)], dstv)
    ones = jnp.ones((16,), F32)

    @pl.loop(0, RT2)
    def _edges(j):
        for b in range(8):
            plsc.addupdate_scatter(acc, [dstv[j, pl.ds(b * 16, 16)]], ones)

    pltpu.sync_copy(acc, out_hbm.at[pl.ds(w * NP, NP)])


@functools.partial(
    pl.kernel,
    out_type=jax.ShapeDtypeStruct((32 * NP,), F32),
    mesh=_MESH,
    scratch_types=[
        pltpu.VMEM((8, 128), jnp.int32),
        pltpu.VMEM((8, 128), jnp.int32),
        pltpu.VMEM((NP,), F32),
        pltpu.VMEM((NP,), F32),
    ],
    compiler_params=pltpu.CompilerParams(needs_layout_passes=False),
)
def _sc_agg_scalar(src_hbm, dst_hbm, y_hbm, out_hbm, srcv, dstv, yv, acc):
    c = lax.axis_index("c")
    s = lax.axis_index("s")
    w = c * 16 + s

    @pl.loop(0, NP // 16)
    def _zero(i):
        acc[pl.ds(i * 16, 16)] = jnp.zeros((16,), F32)

    pltpu.sync_copy(y_hbm, yv)

    @pl.loop(0, RT2 // 8)
    def _chunk(k):
        rsl = pl.ds(w * RT2 + k * 8, 8)
        pltpu.sync_copy(src_hbm.at[rsl], srcv)
        pltpu.sync_copy(dst_hbm.at[rsl], dstv)

        @pl.loop(0, 8)
        def _row(j):
            for b in range(8):
                lane = pl.ds(b * 16, 16)
                vals = plsc.load_gather(yv, [srcv[j, lane]])
                plsc.addupdate_scatter(acc, [dstv[j, lane]], vals)

    pltpu.sync_copy(acc, out_hbm.at[pl.ds(w * NP, NP)])


_QSHAPE = jax.ShapeDtypeStruct((NP, 16), F32)


@functools.partial(
    pl.kernel,
    out_type=(_QSHAPE, _QSHAPE, _QSHAPE, _QSHAPE),
    mesh=_MESH,
    scratch_types=[
        pltpu.VMEM((5120,), jnp.int32),
        pltpu.VMEM((5120,), jnp.int32),
        pltpu.VMEM((1024, 16), F32),
        pltpu.VMEM((1024, 16), F32),
        pltpu.VMEM((136, 16), F32),
        pltpu.SemaphoreType.DMA,
        pltpu.SemaphoreType.DMA,
        pltpu.VMEM_SHARED((NP, 16), F32),
    ],
    compiler_params=pltpu.CompilerParams(use_tc_tiling_on_sc=False),
)
def _sc_agg_wide(z0_hbm, z1_hbm, z2_hbm, z3_hbm, srcf_hbm, dstf_hbm,
                 o0_hbm, o1_hbm, o2_hbm, o3_hbm,
                 srcv, dstv, rowv0, rowv1, bounce, gsem, ssem, acc):
    c = lax.axis_index("c")
    s = lax.axis_index("s")

    def quarter(z_ref, out_ref):
        @pl.loop(0, TS // 136)
        def _init(k):
            csl = pl.ds(s * TS + k * 136, 136)
            pltpu.sync_copy(z_ref.at[csl], bounce)
            pltpu.sync_copy(bounce, acc.at[csl])

        plsc.subcore_barrier()

        @pl.loop(0, RT1 // 40)
        def _sup(k):
            row0 = s * RT1 + k * 40
            pltpu.sync_copy(srcf_hbm.at[pl.ds(row0 * 128, 5120)], srcv)
            pltpu.sync_copy(dstf_hbm.at[pl.ds(row0 * 128, 5120)], dstv)
            bufs = (rowv0, rowv1)

            def fire_gather(t):
                return pltpu.async_copy(
                    z_ref.at[srcv.at[pl.ds(t * 1024, 1024)]],
                    bufs[t % 2], gsem)

            def fire_scatters(t):
                return [pltpu.async_copy(
                    bufs[t % 2], acc.at[dstv.at[pl.ds(t * 1024, 1024)]],
                    ssem, add=True)]

            g = fire_gather(0)
            pend = []
            for t in range(5):
                if t + 1 < 5:
                    if t >= 1:
                        for d in pend:
                            d.wait()
                    g.wait()
                    gn = fire_gather(t + 1)
                else:
                    g.wait()
                    gn = None
                pend_new = fire_scatters(t)
                if t + 1 == 5:
                    for d in pend:
                        d.wait()
                pend = pend_new
                g = gn
            for d in pend:
                d.wait()

        plsc.subcore_barrier()

        @pl.loop(0, TS // 136)
        def _out(k):
            csl = pl.ds(s * TS + k * 136, 136)
            pltpu.sync_copy(acc.at[csl], bounce)
            pltpu.sync_copy(bounce, out_ref.at[csl])

        plsc.subcore_barrier()

    @pl.when(c == 0)
    def _():
        quarter(z0_hbm, o0_hbm)
        quarter(z1_hbm, o1_hbm)

    @pl.when(c == 1)
    def _():
        quarter(z2_hbm, o2_hbm)
        quarter(z3_hbm, o3_hbm)


def _tc_dinv_y(degp, xp):
    def body(dp_ref, x_ref, dinv_ref, y_ref):
        deg = jnp.sum(dp_ref[...], axis=0, keepdims=True) + 1.0
        dinv = lax.rsqrt(deg)
        dinv_ref[...] = dinv
        y_ref[...] = dinv * x_ref[...]

    return pl.pallas_call(
        body,
        out_shape=(jax.ShapeDtypeStruct((1, NP), F32),
                   jax.ShapeDtypeStruct((1, NP), F32)),
    )(degp, xp)


def _tc_s_stats(aggp, y, dinv):
    def body(ap_ref, y_ref, dinv_ref, s_ref, stats_ref):
        agg = jnp.sum(ap_ref[...], axis=0, keepdims=True) + y_ref[...]
        sv = dinv_ref[...] * agg
        s_ref[...] = sv
        col = lax.broadcasted_iota(jnp.int32, (1, NP), 1)
        msk = (col < N).astype(F32)
        sm = jnp.sum(sv * msk)
        sq = jnp.sum(sv * sv * msk)
        m = sm * (1.0 / N)
        v = sq * (1.0 / N) - m * m
        ri = lax.broadcasted_iota(jnp.int32, (8, 128), 0)
        stats_ref[...] = jnp.where(ri == 0, m, v)

    return pl.pallas_call(
        body,
        out_shape=(jax.ShapeDtypeStruct((1, NP), F32),
                   jax.ShapeDtypeStruct((8, 128), F32)),
    )(aggp, y, dinv)


def _full(shape):
    return pl.BlockSpec(shape, lambda i: (0,) * len(shape))


_QBLK = pl.BlockSpec((BN, 16), lambda i: (i, 0))


def _tc_layer1(s, dinv, stats, w1t, g1t, be1t, w2q):
    def body(s_ref, dinv_ref, stats_ref, w1t_ref, g1t_ref, be1t_ref,
             wq0_ref, wq1_ref, wq2_ref, wq3_ref,
             z0_ref, z1_ref, z2_ref, z3_ref):
        m = stats_ref[0:1, 0:1]
        v = stats_ref[1:2, 0:1]
        w1 = w1t_ref[...]
        a = w1 * g1t_ref[...] * lax.rsqrt(v * w1 * w1 + EPS)
        t = s_ref[...] - m
        x1d = jnp.maximum(a * t + be1t_ref[...], 0.0) * dinv_ref[...]
        dn = (((0,), (0,)), ((), ()))
        z0_ref[...] = lax.dot_general(x1d, wq0_ref[...], dn,
                                      preferred_element_type=F32)
        z1_ref[...] = lax.dot_general(x1d, wq1_ref[...], dn,
                                      preferred_element_type=F32)
        z2_ref[...] = lax.dot_general(x1d, wq2_ref[...], dn,
                                      preferred_element_type=F32)
        z3_ref[...] = lax.dot_general(x1d, wq3_ref[...], dn,
                                      preferred_element_type=F32)

    return pl.pallas_call(
        body,
        grid=(GRID,),
        in_specs=[
            pl.BlockSpec((1, BN), lambda i: (0, i)),
            pl.BlockSpec((1, BN), lambda i: (0, i)),
            _full((8, 128)),
            _full((256, 1)), _full((256, 1)), _full((256, 1)),
            _full((256, 16)), _full((256, 16)), _full((256, 16)),
            _full((256, 16)),
        ],
        out_specs=(_QBLK, _QBLK, _QBLK, _QBLK),
        out_shape=(_QSHAPE, _QSHAPE, _QSHAPE, _QSHAPE),
    )(s, dinv, stats, w1t, g1t, be1t, *w2q)


def _tc_stats2(aggq, dinv16, b2q):
    def body(a0_ref, a1_ref, a2_ref, a3_ref, dinv_ref,
             b0_ref, b1_ref, b2_ref, b3_ref,
             s0_ref, s1_ref, s2_ref, s3_ref):
        i = pl.program_id(0)
        row = lax.broadcasted_iota(jnp.int32, (BN, 16), 0) + i * BN
        msk = (row < N).astype(F32)
        d = dinv_ref[...]
        ri = lax.broadcasted_iota(jnp.int32, (8, 16), 0)
        for a_ref, b_ref, s_ref in ((a0_ref, b0_ref, s0_ref),
                                    (a1_ref, b1_ref, s1_ref),
                                    (a2_ref, b2_ref, s2_ref),
                                    (a3_ref, b3_ref, s3_ref)):
            x = (d * a_ref[...] + b_ref[...]) * msk
            blk = jnp.where(ri == 0, jnp.sum(x, axis=0, keepdims=True),
                            jnp.sum(x * x, axis=0, keepdims=True))

            @pl.when(i == 0)
            def _(s_ref=s_ref, blk=blk):
                s_ref[...] = blk

            @pl.when(i > 0)
            def _(s_ref=s_ref, blk=blk):
                s_ref[...] += blk

    stat_shape = jax.ShapeDtypeStruct((8, 16), F32)
    stat_blk = pl.BlockSpec((8, 16), lambda i: (0, 0))
    return pl.pallas_call(
        body,
        grid=(GRID,),
        in_specs=[_QBLK, _QBLK, _QBLK, _QBLK, _QBLK,
                  _full((1, 16)), _full((1, 16)), _full((1, 16)),
                  _full((1, 16))],
        out_specs=(stat_blk,) * 4,
        out_shape=(stat_shape,) * 4,
    )(*aggq, dinv16, *b2q)


def _tc_layer2(aggq, dinv16, b2q, statq, g2q, be2q, w3q):
    def body(a0_ref, a1_ref, a2_ref, a3_ref, dinv_ref,
             b0_ref, b1_ref, b2_ref, b3_ref,
             s0_ref, s1_ref, s2_ref, s3_ref,
             g0_ref, g1_ref, g2_ref, g3_ref,
             e0_ref, e1_ref, e2_ref, e3_ref,
             w0_ref, w1_ref, w2_ref, w3_ref,
             z0_ref, z1_ref, z2_ref, z3_ref):
        d = dinv_ref[...]
        inv_n = 1.0 / N

        def norm(a_ref, b_ref, s_ref, g_ref, e_ref):
            xpre = d * a_ref[...] + b_ref[...]
            mu = s_ref[0:1, :] * inv_n
            var = s_ref[1:2, :] * inv_n - mu * mu
            return jnp.maximum(
                (xpre - mu) * lax.rsqrt(var + EPS) * g_ref[...] + e_ref[...],
                0.0)

        x2 = jnp.concatenate([
            norm(a0_ref, b0_ref, s0_ref, g0_ref, e0_ref),
            norm(a1_ref, b1_ref, s1_ref, g1_ref, e1_ref),
            norm(a2_ref, b2_ref, s2_ref, g2_ref, e2_ref),
            norm(a3_ref, b3_ref, s3_ref, g3_ref, e3_ref)], axis=1)
        for w_ref, z_ref in ((w0_ref, z0_ref), (w1_ref, z1_ref),
                             (w2_ref, z2_ref), (w3_ref, z3_ref)):
            z_ref[...] = d * jnp.dot(x2, w_ref[...],
                                     preferred_element_type=F32)

    stat_blk = pl.BlockSpec((8, 16), lambda i: (0, 0))
    return pl.pallas_call(
        body,
        grid=(GRID,),
        in_specs=[_QBLK, _QBLK, _QBLK, _QBLK, _QBLK,
                  _full((1, 16)), _full((1, 16)), _full((1, 16)),
                  _full((1, 16)),
                  stat_blk, stat_blk, stat_blk, stat_blk,
                  _full((1, 16)), _full((1, 16)), _full((1, 16)),
                  _full((1, 16)),
                  _full((1, 16)), _full((1, 16)), _full((1, 16)),
                  _full((1, 16)),
                  _full((64, 16)), _full((64, 16)), _full((64, 16)),
                  _full((64, 16))],
        out_specs=(_QBLK, _QBLK, _QBLK, _QBLK),
        out_shape=(_QSHAPE, _QSHAPE, _QSHAPE, _QSHAPE),
    )(*aggq, dinv16, *b2q, *statq, *g2q, *be2q, *w3q)


def _tc_final(aggq, dinv16, b3q):
    def body(a0_ref, a1_ref, a2_ref, a3_ref, dinv_ref,
             b0_ref, b1_ref, b2_ref, b3_ref, o_ref):
        d = dinv_ref[...]
        o_ref[...] = jnp.concatenate(
            [jnp.maximum(d * a_ref[...] + b_ref[...], 0.0)
             for a_ref, b_ref in ((a0_ref, b0_ref), (a1_ref, b1_ref),
                                  (a2_ref, b2_ref), (a3_ref, b3_ref))],
            axis=1)

    return pl.pallas_call(
        body,
        grid=(GRID,),
        in_specs=[_QBLK, _QBLK, _QBLK, _QBLK, _QBLK,
                  _full((1, 16)), _full((1, 16)), _full((1, 16)),
                  _full((1, 16))],
        out_specs=pl.BlockSpec((BN, 64), lambda i: (i, 0)),
        out_shape=jax.ShapeDtypeStruct((N, 64), F32),
    )(*aggq, dinv16, *b3q)


def kernel(data, edge_index, W1, b1, g1, be1, W2, b2, g2, be2, W3, b3):
    padn = EP - E
    fill_src = lax.iota(jnp.int32, padn) % 128
    fill_dst = N + lax.iota(jnp.int32, padn) % (NP - N)
    src2d = jnp.concatenate([edge_index[0], fill_src]).reshape(ERP, 128)
    dst2d = jnp.concatenate([edge_index[1], fill_dst]).reshape(ERP, 128)
    xp = jnp.pad(data[:, 0], (0, NP - N)).reshape(1, NP)

    w1t = W1.reshape(256, 1)
    g1t = g1.reshape(256, 1)
    be1t = be1.reshape(256, 1)
    quarters = lambda v: [v[..., 16 * q:16 * (q + 1)] for q in range(4)]
    w2q = quarters(W2)
    b2q = [b.reshape(1, 16) for b in quarters(b2)]
    g2q = [b.reshape(1, 16) for b in quarters(g2)]
    be2q = [b.reshape(1, 16) for b in quarters(be2)]
    w3q = quarters(W3)
    b3q = [b.reshape(1, 16) for b in quarters(b3)]

    degp = _sc_deg(dst2d)
    dinv, y = _tc_dinv_y(degp.reshape(32, NP), xp)
    aggp = _sc_agg_scalar(src2d, dst2d, y.reshape(NP))
    s, stats = _tc_s_stats(aggp.reshape(32, NP), y, dinv)
    z2q = _tc_layer1(s, dinv, stats, w1t, g1t, be1t, w2q)
    dinv16 = jnp.broadcast_to(dinv.reshape(NP, 1), (NP, 16))
    srcf = src2d.reshape(EP)
    dstf = dst2d.reshape(EP)
    a2q = _sc_agg_wide(*z2q, srcf, dstf)
    statq = _tc_stats2(a2q, dinv16, b2q)
    z3q = _tc_layer2(a2q, dinv16, b2q, statq, g2q, be2q, w3q)
    a3q = _sc_agg_wide(*z3q, srcf, dstf)
    return _tc_final(a3q, dinv16, b3q)
